# Initial kernel scaffold; baseline (speedup 1.0000x reference)
import jax
import jax.numpy as jnp
from jax.experimental import pallas as pl


def kernel(input, weight):
    raise NotImplementedError("write your pallas kernel here")


# speedup vs baseline: 1.4598x; 1.4598x over previous
import functools

import jax
import jax.numpy as jnp
from jax import lax
from jax.experimental import pallas as pl
from jax.experimental.pallas import tpu as pltpu
from jax.experimental.pallas import tpu_sc as plsc

BATCH = 4096
SEQ_LEN = 200
EMBEDDING_DIM = 32

NUM_WORKERS = 32
CHUNK = 128
GATHERS_PER_SUPER = 8
SUPER = CHUNK * GATHERS_PER_SUPER
B_TOTAL = BATCH * SEQ_LEN
PER_W = B_TOTAL // NUM_WORKERS
N_SUPER = PER_W // SUPER
IDX_ROWS_PER_SUPER = SUPER // CHUNK


def _gather_kernel(idx_hbm, table_hbm, out_hbm, idx_v, rows_v, gsem):
    c = lax.axis_index("c")
    s = lax.axis_index("s")
    wid = s * 2 + c
    row_base = wid * (PER_W // CHUNK)

    def step(i, carry):
        idx_row = row_base + i * IDX_ROWS_PER_SUPER
        out_off = wid * PER_W + i * SUPER
        pltpu.sync_copy(idx_hbm.at[pl.ds(idx_row, IDX_ROWS_PER_SUPER)], idx_v)
        copies = []
        for j in range(GATHERS_PER_SUPER):
            copies.append(
                pltpu.async_copy(
                    table_hbm.at[idx_v.at[j]],
                    rows_v.at[pl.ds(j * CHUNK, CHUNK)],
                    gsem,
                )
            )
        for cp in copies:
            cp.wait()
        pltpu.sync_copy(rows_v, out_hbm.at[pl.ds(out_off, SUPER)])
        return carry

    lax.fori_loop(0, N_SUPER, step, 0)


def kernel(input, weight):
    idx2d = input.reshape(B_TOTAL // CHUNK, CHUNK)
    mesh = plsc.VectorSubcoreMesh(core_axis_name="c", subcore_axis_name="s")
    run = functools.partial(
        pl.kernel,
        mesh=mesh,
        out_type=jax.ShapeDtypeStruct((B_TOTAL, EMBEDDING_DIM), jnp.float32),
        scratch_types=[
            pltpu.VMEM((IDX_ROWS_PER_SUPER, CHUNK), jnp.int32),
            pltpu.VMEM((SUPER, EMBEDDING_DIM), jnp.float32),
            pltpu.SemaphoreType.DMA,
        ],
        compiler_params=pltpu.CompilerParams(use_tc_tiling_on_sc=False),
    )(_gather_kernel)
    out = run(idx2d, weight)
    return out.reshape(BATCH, SEQ_LEN, EMBEDDING_DIM)


# speedup vs baseline: 1.4933x; 1.0230x over previous
import functools

import jax
import jax.numpy as jnp
from jax import lax
from jax.experimental import pallas as pl
from jax.experimental.pallas import tpu as pltpu
from jax.experimental.pallas import tpu_sc as plsc

BATCH = 4096
SEQ_LEN = 200
EMBEDDING_DIM = 32

NUM_WORKERS = 32
CHUNK = 128
GATHERS_PER_SUPER = 10
SUPER = CHUNK * GATHERS_PER_SUPER
B_TOTAL = BATCH * SEQ_LEN
PER_W = B_TOTAL // NUM_WORKERS
N_SUPER = PER_W // SUPER
IDX_ROWS = PER_W // CHUNK


def _gather_kernel(idx_hbm, table_hbm, out_hbm, idx_v, rows_v, gsem, wsem):
    c = lax.axis_index("c")
    s = lax.axis_index("s")
    wid = s * 2 + c
    out_base = wid * PER_W
    pltpu.sync_copy(idx_hbm.at[pl.ds(wid * IDX_ROWS, IDX_ROWS)], idx_v)

    def drain_write(b):
        pltpu.make_async_copy(out_hbm.at[pl.ds(0, SUPER)], rows_v.at[b], wsem).wait()

    def do_chunk(chunk, b):
        cps = [
            pltpu.async_copy(
                table_hbm.at[idx_v.at[chunk * GATHERS_PER_SUPER + j]],
                rows_v.at[b].at[pl.ds(j * CHUNK, CHUNK)],
                gsem,
            )
            for j in range(GATHERS_PER_SUPER)
        ]
        for cp in cps:
            cp.wait()
        pltpu.async_copy(
            rows_v.at[b],
            out_hbm.at[pl.ds(out_base + chunk * SUPER, SUPER)],
            wsem,
        )

    def body(outer, carry):
        @pl.when(outer >= 1)
        def _():
            drain_write(0)

        do_chunk(2 * outer, 0)

        @pl.when(outer >= 1)
        def _():
            drain_write(1)

        do_chunk(2 * outer + 1, 1)
        return carry

    lax.fori_loop(0, N_SUPER // 2, body, 0)
    drain_write(0)
    drain_write(1)


def kernel(input, weight):
    idx2d = input.reshape(B_TOTAL // CHUNK, CHUNK)
    mesh = plsc.VectorSubcoreMesh(core_axis_name="c", subcore_axis_name="s")
    run = functools.partial(
        pl.kernel,
        mesh=mesh,
        out_type=jax.ShapeDtypeStruct((B_TOTAL, EMBEDDING_DIM), jnp.float32),
        scratch_types=[
            pltpu.VMEM((IDX_ROWS, CHUNK), jnp.int32),
            pltpu.VMEM((2, SUPER, EMBEDDING_DIM), jnp.float32),
            pltpu.SemaphoreType.DMA,
            pltpu.SemaphoreType.DMA,
        ],
        compiler_params=pltpu.CompilerParams(use_tc_tiling_on_sc=False),
    )(_gather_kernel)
    out = run(idx2d, weight)
    return out.reshape(BATCH, SEQ_LEN, EMBEDDING_DIM)


# speedup vs baseline: 1.5015x; 1.0055x over previous
import functools

import jax
import jax.numpy as jnp
from jax import lax
from jax.experimental import pallas as pl
from jax.experimental.pallas import tpu as pltpu
from jax.experimental.pallas import tpu_sc as plsc

BATCH = 4096
SEQ_LEN = 200
EMBEDDING_DIM = 32

NUM_WORKERS = 32
CHUNK = 128
GATHERS_PER_SUPER = 10
SUPER = CHUNK * GATHERS_PER_SUPER
B_TOTAL = BATCH * SEQ_LEN
PER_W = B_TOTAL // NUM_WORKERS
N_SUPER = PER_W // SUPER
IDX_ROWS = PER_W // CHUNK


def _gather_kernel(idx_hbm, table_hbm, out_hbm, idx_v, rows_v, g0, g1, w0, w1):
    gsems = (g0, g1)
    wsems = (w0, w1)
    c = lax.axis_index("c")
    s = lax.axis_index("s")
    wid = s * 2 + c
    out_base = wid * PER_W
    pltpu.sync_copy(idx_hbm.at[pl.ds(wid * IDX_ROWS, IDX_ROWS)], idx_v)

    def issue_gathers(chunk, b):
        for j in range(GATHERS_PER_SUPER):
            pltpu.async_copy(
                table_hbm.at[idx_v.at[chunk * GATHERS_PER_SUPER + j]],
                rows_v.at[b].at[pl.ds(j * CHUNK, CHUNK)],
                gsems[b],
            )

    def drain_gathers(b):
        pltpu.make_async_copy(out_hbm.at[pl.ds(0, SUPER)], rows_v.at[b], gsems[b]).wait()

    def start_write(chunk, b):
        pltpu.async_copy(
            rows_v.at[b],
            out_hbm.at[pl.ds(out_base + chunk * SUPER, SUPER)],
            wsems[b],
        )

    def drain_write(b):
        pltpu.make_async_copy(out_hbm.at[pl.ds(0, SUPER)], rows_v.at[b], wsems[b]).wait()

    issue_gathers(0, 0)

    def body(outer, carry):
        @pl.when(outer >= 1)
        def _():
            drain_write(1)

        issue_gathers(2 * outer + 1, 1)
        drain_gathers(0)
        start_write(2 * outer, 0)

        @pl.when(outer + 1 < N_SUPER // 2)
        def _():
            drain_write(0)
            issue_gathers(2 * outer + 2, 0)

        drain_gathers(1)
        start_write(2 * outer + 1, 1)
        return carry

    lax.fori_loop(0, N_SUPER // 2, body, 0)
    drain_write(0)
    drain_write(1)


def kernel(input, weight):
    idx2d = input.reshape(B_TOTAL // CHUNK, CHUNK)
    mesh = plsc.VectorSubcoreMesh(core_axis_name="c", subcore_axis_name="s")
    run = functools.partial(
        pl.kernel,
        mesh=mesh,
        out_type=jax.ShapeDtypeStruct((B_TOTAL, EMBEDDING_DIM), jnp.float32),
        scratch_types=[
            pltpu.VMEM((IDX_ROWS, CHUNK), jnp.int32),
            pltpu.VMEM((2, SUPER, EMBEDDING_DIM), jnp.float32),
            pltpu.SemaphoreType.DMA,
            pltpu.SemaphoreType.DMA,
            pltpu.SemaphoreType.DMA,
            pltpu.SemaphoreType.DMA,
        ],
        compiler_params=pltpu.CompilerParams(use_tc_tiling_on_sc=False),
    )(_gather_kernel)
    out = run(idx2d, weight)
    return out.reshape(BATCH, SEQ_LEN, EMBEDDING_DIM)
